# baseline (speedup 1.0000x reference)
import jax
import jax.numpy as jnp
from jax.experimental import pallas as pl
from jax.experimental.pallas import tpu as pltpu

_BLOCK_ROWS = 2048


def _net_kernel(x_ref, w1_ref, b1_ref, w2_ref, b2_ref, o_ref):
    wf = jnp.dot(w1_ref[...], w2_ref[...], preferred_element_type=jnp.float32)
    bf = jnp.dot(b1_ref[...], w2_ref[...], preferred_element_type=jnp.float32) + b2_ref[...]
    o_ref[...] = jnp.dot(x_ref[...], wf, preferred_element_type=jnp.float32) + bf


def kernel(x, edge_index, W1, b1, W2, b2):
    n, in_dim = x.shape
    hid = W1.shape[1]
    out_dim = W2.shape[1]
    b1_2d = b1.reshape(1, hid)
    b2_2d = b2.reshape(1, out_dim)
    return pl.pallas_call(
        _net_kernel,
        grid=(pl.cdiv(n, _BLOCK_ROWS),),
        in_specs=[
            pl.BlockSpec((_BLOCK_ROWS, in_dim), lambda i: (i, 0)),
            pl.BlockSpec((in_dim, hid), lambda i: (0, 0)),
            pl.BlockSpec((1, hid), lambda i: (0, 0)),
            pl.BlockSpec((hid, out_dim), lambda i: (0, 0)),
            pl.BlockSpec((1, out_dim), lambda i: (0, 0)),
        ],
        out_specs=pl.BlockSpec((_BLOCK_ROWS, out_dim), lambda i: (i, 0)),
        out_shape=jax.ShapeDtypeStruct((n, out_dim), x.dtype),
        compiler_params=pltpu.CompilerParams(dimension_semantics=("parallel",)),
    )(x, W1, b1_2d, W2, b2_2d)


# speedup vs baseline: 1.1395x; 1.1395x over previous
import jax
import jax.numpy as jnp
from jax.experimental import pallas as pl
from jax.experimental.pallas import tpu as pltpu

_BLOCK_ROWS = 2048


def _net_kernel(x_ref, w1_ref, b1_ref, w2_ref, b2_ref, o_ref):
    wf = jnp.dot(w1_ref[...], w2_ref[...], preferred_element_type=jnp.float32)
    bf = jnp.dot(b1_ref[...], w2_ref[...], preferred_element_type=jnp.float32) + b2_ref[...]
    o_ref[...] = jnp.dot(x_ref[...], wf, preferred_element_type=jnp.float32) + bf


def kernel(x, edge_index, W1, b1, W2, b2):
    n, in_dim = x.shape
    hid = W1.shape[1]
    out_dim = W2.shape[1]
    b1_2d = b1.reshape(1, hid)
    b2_2d = b2.reshape(1, out_dim)
    return pl.pallas_call(
        _net_kernel,
        out_shape=jax.ShapeDtypeStruct((n, out_dim), x.dtype),
    )(x, W1, b1_2d, W2, b2_2d)
